# baseline (speedup 1.0000x reference)
import functools

import jax
import jax.numpy as jnp
from jax import lax
from jax.experimental import pallas as pl
from jax.experimental.pallas import tpu as pltpu

_TOP_K = 2
_ALPHA = 0.01
_TB = 2048


def _router_body(x_ref, w_ref, wout_ref, iout_ref, stats_ref,
                 logit_buf, p_acc, c_acc, *, n_tokens, n_experts, n_blocks):
    step = pl.program_id(0)

    prev = logit_buf[...]

    x = x_ref[...]
    w = w_ref[...]
    logit_buf[...] = lax.dot_general(
        w, x, (((1,), (1,)), ((), ())), preferred_element_type=jnp.float32
    )

    eidx = lax.broadcasted_iota(jnp.int32, prev.shape, 0)
    m1 = jnp.max(prev, axis=0, keepdims=True)
    i1 = jnp.min(jnp.where(prev == m1, eidx, n_experts), axis=0, keepdims=True)
    masked = jnp.where(eidx == i1, -jnp.inf, prev)
    m2 = jnp.max(masked, axis=0, keepdims=True)
    i2 = jnp.min(jnp.where(masked == m2, eidx, n_experts), axis=0, keepdims=True)

    ex = jnp.exp(prev - m1)
    z = jnp.sum(ex, axis=0, keepdims=True)
    p1 = 1.0 / z
    p2 = jnp.exp(m2 - m1) / z
    denom = p1 + p2 + 1e-9
    wout_ref[...] = jnp.concatenate([p1 / denom, p2 / denom], axis=0).T
    iout_ref[...] = jnp.concatenate([i1, i2], axis=0).T

    one_hot = (eidx == i1).astype(jnp.float32) + (eidx == i2).astype(jnp.float32)
    p_acc[...] += ex / z
    c_acc[...] += one_hot

    @pl.when(step == 0)
    def _init():
        p_acc[...] = jnp.zeros_like(p_acc)
        c_acc[...] = jnp.zeros_like(c_acc)

    @pl.when(step == n_blocks)
    def _finish():
        p_mean = jnp.sum(p_acc[...], axis=1, keepdims=True) / n_tokens
        f_mean = jnp.sum(c_acc[...], axis=1, keepdims=True) / (n_tokens * _TOP_K)
        aux = _ALPHA * n_experts * jnp.sum(p_mean * f_mean)
        stats_ref[...] = jnp.broadcast_to(aux, (1, n_experts))


def kernel(hidden_states, gate_weight):
    b, s, h = hidden_states.shape
    e = gate_weight.shape[0]
    t = b * s
    n_blocks = t // _TB

    x = hidden_states.reshape(t, h)
    body = functools.partial(
        _router_body, n_tokens=t, n_experts=e, n_blocks=n_blocks)
    wout, iout, stats = pl.pallas_call(
        body,
        grid=(n_blocks + 1,),
        in_specs=[
            pl.BlockSpec((_TB, h), lambda i: (jnp.minimum(i, n_blocks - 1), 0)),
            pl.BlockSpec((e, h), lambda i: (0, 0)),
        ],
        out_specs=[
            pl.BlockSpec((_TB, _TOP_K), lambda i: (jnp.maximum(i - 1, 0), 0)),
            pl.BlockSpec((_TB, _TOP_K), lambda i: (jnp.maximum(i - 1, 0), 0)),
            pl.BlockSpec((1, e), lambda i: (0, 0)),
        ],
        out_shape=[
            jax.ShapeDtypeStruct((t, _TOP_K), jnp.float32),
            jax.ShapeDtypeStruct((t, _TOP_K), jnp.int32),
            jax.ShapeDtypeStruct((1, e), jnp.float32),
        ],
        scratch_shapes=[
            pltpu.VMEM((e, _TB), jnp.float32),
            pltpu.VMEM((e, _TB), jnp.float32),
            pltpu.VMEM((e, _TB), jnp.float32),
        ],
    )(x, gate_weight)

    return (
        wout.reshape(b, s, _TOP_K),
        iout.reshape(b, s, _TOP_K).astype(jnp.int64),
        stats[0, 0],
    )


# speedup vs baseline: 1.0443x; 1.0443x over previous
import functools

import jax
import jax.numpy as jnp
from jax import lax
from jax.experimental import pallas as pl
from jax.experimental.pallas import tpu as pltpu

_TOP_K = 2
_ALPHA = 0.01
_TB = 1024


def _router_body(x_ref, w_ref, wout_ref, iout_ref, stats_ref,
                 logit_buf, p_acc, c_acc, *, n_tokens, n_experts, n_blocks):
    step = pl.program_id(0)

    prev = logit_buf[...]

    x = x_ref[...]
    w = w_ref[...]
    logit_buf[...] = lax.dot_general(
        w, x, (((1,), (1,)), ((), ())), preferred_element_type=jnp.float32
    )

    eidx = lax.broadcasted_iota(jnp.int32, prev.shape, 0)
    m1 = jnp.max(prev, axis=0, keepdims=True)
    i1 = jnp.min(jnp.where(prev == m1, eidx, n_experts), axis=0, keepdims=True)
    masked = jnp.where(eidx == i1, -jnp.inf, prev)
    m2 = jnp.max(masked, axis=0, keepdims=True)
    i2 = jnp.min(jnp.where(masked == m2, eidx, n_experts), axis=0, keepdims=True)

    ex = jnp.exp(prev - m1)
    z = jnp.sum(ex, axis=0, keepdims=True)
    p1 = 1.0 / z
    p2 = jnp.exp(m2 - m1) / z
    denom = p1 + p2 + 1e-9
    wout_ref[...] = jnp.concatenate([p1 / denom, p2 / denom], axis=0).T
    iout_ref[...] = jnp.concatenate([i1, i2], axis=0).T

    one_hot = (eidx == i1).astype(jnp.float32) + (eidx == i2).astype(jnp.float32)
    p_acc[...] += ex / z
    c_acc[...] += one_hot

    @pl.when(step == 0)
    def _init():
        p_acc[...] = jnp.zeros_like(p_acc)
        c_acc[...] = jnp.zeros_like(c_acc)

    @pl.when(step == n_blocks)
    def _finish():
        p_mean = jnp.sum(p_acc[...], axis=1, keepdims=True) / n_tokens
        f_mean = jnp.sum(c_acc[...], axis=1, keepdims=True) / (n_tokens * _TOP_K)
        aux = _ALPHA * n_experts * jnp.sum(p_mean * f_mean)
        stats_ref[...] = jnp.broadcast_to(aux, (1, n_experts))


def kernel(hidden_states, gate_weight):
    b, s, h = hidden_states.shape
    e = gate_weight.shape[0]
    t = b * s
    n_blocks = t // _TB

    x = hidden_states.reshape(t, h)
    body = functools.partial(
        _router_body, n_tokens=t, n_experts=e, n_blocks=n_blocks)
    wout, iout, stats = pl.pallas_call(
        body,
        grid=(n_blocks + 1,),
        in_specs=[
            pl.BlockSpec((_TB, h), lambda i: (jnp.minimum(i, n_blocks - 1), 0)),
            pl.BlockSpec((e, h), lambda i: (0, 0)),
        ],
        out_specs=[
            pl.BlockSpec((_TB, _TOP_K), lambda i: (jnp.maximum(i - 1, 0), 0)),
            pl.BlockSpec((_TB, _TOP_K), lambda i: (jnp.maximum(i - 1, 0), 0)),
            pl.BlockSpec((1, e), lambda i: (0, 0)),
        ],
        out_shape=[
            jax.ShapeDtypeStruct((t, _TOP_K), jnp.float32),
            jax.ShapeDtypeStruct((t, _TOP_K), jnp.int32),
            jax.ShapeDtypeStruct((1, e), jnp.float32),
        ],
        scratch_shapes=[
            pltpu.VMEM((e, _TB), jnp.float32),
            pltpu.VMEM((e, _TB), jnp.float32),
            pltpu.VMEM((e, _TB), jnp.float32),
        ],
    )(x, gate_weight)

    return (
        wout.reshape(b, s, _TOP_K),
        iout.reshape(b, s, _TOP_K).astype(jnp.int64),
        stats[0, 0],
    )
